# Initial kernel scaffold; baseline (speedup 1.0000x reference)
import jax
import jax.numpy as jnp
from jax import lax
from jax.experimental import pallas as pl
from jax.experimental.pallas import tpu as pltpu
from jax.experimental.pallas import tpu_sc as plsc

NUM_USERS = 50000
NUM_ITEMS = 50000
N = NUM_USERS + NUM_ITEMS
H = 16
NC = 2
NS = 16

NODES_PER_TILE = 6272
NPAD = NS * NODES_PER_TILE
NCHUNK = 448
NODE_CHUNKS = NODES_PER_TILE // NCHUNK

E = 1600000
ROWS_PER_CHUNK = 8
EDGE_CHUNKS = 98
ROWS_PER_TILE = EDGE_CHUNKS * ROWS_PER_CHUNK
EP = NS * ROWS_PER_TILE * 128

_F32 = jnp.float32
_I32 = jnp.int32


def _rsqrt16(d):
    i = plsc.bitcast(d, _I32)
    y = plsc.bitcast(jnp.int32(0x5F3759DF) - (i >> 1), _F32)
    half = d * 0.5
    for _ in range(3):
        y = y * (1.5 - half * y * y)
    return jnp.where(d > 0.5, y, jnp.zeros_like(y))


def _gcn_body(colp, rowp, x0p,
              s_out, y0, y1, y2, disx,
              acc, dacc,
              cbuf, rbuf, gbuf, onesb, zbuf, z1d,
              d1d, r1d, debuf, xbuf, abuf, sbuf):
    c = lax.axis_index("c")
    t = lax.axis_index("s")
    coff = c * NPAD
    nbase0 = t * NODES_PER_TILE
    rbase0 = t * ROWS_PER_TILE

    onesb[...] = jnp.ones((ROWS_PER_CHUNK, 128), _F32)
    zbuf[...] = jnp.zeros((NCHUNK, H), _F32)
    z1d[...] = jnp.zeros((NCHUNK,), _F32)

    @pl.loop(0, NODE_CHUNKS)
    def _zero(q):
        nb = nbase0 + q * NCHUNK
        pltpu.sync_copy(zbuf, acc.at[pl.ds(nb, NCHUNK)])
        pltpu.sync_copy(z1d, dacc.at[pl.ds(nb, NCHUNK)])

    plsc.subcore_barrier()

    @pl.loop(0, EDGE_CHUNKS)
    def _deg(m):
        rb = rbase0 + m * ROWS_PER_CHUNK
        pltpu.sync_copy(colp.at[pl.ds(rb, ROWS_PER_CHUNK)], cbuf)
        pltpu.sync_copy(onesb, dacc.at[cbuf], add=True)

    plsc.subcore_barrier()

    @pl.loop(0, NODE_CHUNKS)
    def _prep(q):
        nb = nbase0 + q * NCHUNK
        pltpu.sync_copy(dacc.at[pl.ds(nb, NCHUNK)], d1d)

        @pl.loop(0, NCHUNK // 16)
        def _r(i):
            d = d1d[pl.ds(i * 16, 16)]
            r1d[pl.ds(i * 16, 16)] = _rsqrt16(d)

        @pl.loop(0, NCHUNK)
        def _b(n):
            debuf[n, :] = jnp.full((H,), r1d[n], _F32)

        pltpu.sync_copy(x0p.at[pl.ds(nb, NCHUNK), pl.ds(c * H, H)], xbuf)

        @pl.loop(0, NCHUNK)
        def _y(n):
            xh = xbuf[n, :]
            sbuf[n, :] = xh
            abuf[n, :] = debuf[n, :] * xh

        pltpu.sync_copy(debuf, disx.at[c, pl.ds(nb, NCHUNK)])
        pltpu.sync_copy(sbuf, s_out.at[pl.ds(coff + nb, NCHUNK)])
        pltpu.sync_copy(abuf, y0.at[pl.ds(coff + nb, NCHUNK)])

    plsc.subcore_barrier()

    for ysrc, ydst in [(y0, y1), (y1, y2), (y2, None)]:
        last = ydst is None

        @pl.loop(0, EDGE_CHUNKS)
        def _edges(m):
            rb = rbase0 + m * ROWS_PER_CHUNK
            pltpu.sync_copy(colp.at[pl.ds(rb, ROWS_PER_CHUNK)], cbuf)

            @pl.loop(0, ROWS_PER_CHUNK)
            def _off(r):
                @pl.loop(0, 8)
                def _offv(v):
                    sl = pl.ds(v * 16, 16)
                    cbuf[r, sl] = cbuf[r, sl] + coff

            pltpu.sync_copy(rowp.at[pl.ds(rb, ROWS_PER_CHUNK)], rbuf)
            pltpu.sync_copy(ysrc.at[cbuf], gbuf)
            pltpu.sync_copy(gbuf, acc.at[rbuf], add=True)

        plsc.subcore_barrier()

        @pl.loop(0, NODE_CHUNKS)
        def _post(q):
            nb = nbase0 + q * NCHUNK
            pltpu.sync_copy(acc.at[pl.ds(nb, NCHUNK)], abuf)
            if not last:
                pltpu.sync_copy(zbuf, acc.at[pl.ds(nb, NCHUNK)])
            pltpu.sync_copy(disx.at[c, pl.ds(nb, NCHUNK)], debuf)
            pltpu.sync_copy(s_out.at[pl.ds(coff + nb, NCHUNK)], sbuf)

            @pl.loop(0, NCHUNK)
            def _n(n):
                d = debuf[n, :]
                x = d * abuf[n, :]
                if last:
                    sbuf[n, :] = (sbuf[n, :] + x) * 0.25
                else:
                    sbuf[n, :] = sbuf[n, :] + x
                    abuf[n, :] = d * x

            pltpu.sync_copy(sbuf, s_out.at[pl.ds(coff + nb, NCHUNK)])
            if not last:
                pltpu.sync_copy(abuf, ydst.at[pl.ds(coff + nb, NCHUNK)])

        plsc.subcore_barrier()


@jax.jit
def _lightgcn(colp, rowp, x0p):
    mesh = plsc.VectorSubcoreMesh(core_axis_name="c", subcore_axis_name="s",
                                  num_cores=NC, num_subcores=NS)
    f = pl.kernel(
        _gcn_body,
        out_type=(
            jax.ShapeDtypeStruct((2 * NPAD, H), _F32),
            jax.ShapeDtypeStruct((2 * NPAD, H), _F32),
            jax.ShapeDtypeStruct((2 * NPAD, H), _F32),
            jax.ShapeDtypeStruct((2 * NPAD, H), _F32),
            jax.ShapeDtypeStruct((NC, NPAD, H), _F32),
        ),
        mesh=mesh,
        scratch_types=[
            pltpu.VMEM_SHARED((NPAD, H), _F32),
            pltpu.VMEM_SHARED((NPAD,), _F32),
            pltpu.VMEM((ROWS_PER_CHUNK, 128), _I32),
            pltpu.VMEM((ROWS_PER_CHUNK, 128), _I32),
            pltpu.VMEM((ROWS_PER_CHUNK, 128, H), _F32),
            pltpu.VMEM((ROWS_PER_CHUNK, 128), _F32),
            pltpu.VMEM((NCHUNK, H), _F32),
            pltpu.VMEM((NCHUNK,), _F32),
            pltpu.VMEM((NCHUNK,), _F32),
            pltpu.VMEM((NCHUNK,), _F32),
            pltpu.VMEM((NCHUNK, H), _F32),
            pltpu.VMEM((NCHUNK, H), _F32),
            pltpu.VMEM((NCHUNK, H), _F32),
            pltpu.VMEM((NCHUNK, H), _F32),
        ],
    )
    return f(colp, rowp, x0p)


def kernel(edge_index, user_weight, item_weight):
    ei = edge_index.astype(_I32)
    pad = N + (jnp.arange(EP - E, dtype=_I32) % 16)
    rowp = jnp.concatenate([ei[0], pad]).reshape(EP // 128, 128)
    colp = jnp.concatenate([ei[1], pad]).reshape(EP // 128, 128)
    x0 = jnp.concatenate([user_weight, item_weight], axis=0)
    x0p = jnp.concatenate([x0, jnp.zeros((NPAD - N, 32), _F32)], axis=0)
    s, _, _, _, _ = _lightgcn(colp, rowp, x0p)
    final = jnp.concatenate([s[:N], s[NPAD:NPAD + N]], axis=1)
    return final[:NUM_USERS], final[NUM_USERS:]


# speedup vs baseline: 19.2834x; 19.2834x over previous
import jax
import jax.numpy as jnp
from jax import lax
from jax.experimental import pallas as pl
from jax.experimental.pallas import tpu as pltpu
from jax.experimental.pallas import tpu_sc as plsc

NUM_USERS = 50000
NUM_ITEMS = 50000
N = NUM_USERS + NUM_ITEMS
H = 16
NC = 2
NS = 16

NODES_PER_TILE = 6272
NPAD = NS * NODES_PER_TILE
NCHUNK = 112
NODE_CHUNKS = NODES_PER_TILE // NCHUNK

E = 1600000
ROWS_PER_CHUNK = 4
EDGE_CHUNKS = 196
ROWS_PER_TILE = EDGE_CHUNKS * ROWS_PER_CHUNK
EP = NS * ROWS_PER_TILE * 128

_F32 = jnp.float32
_I32 = jnp.int32


def _rsqrt16(d):
    i = lax.bitcast_convert_type(d, _I32)
    y = lax.bitcast_convert_type(jnp.int32(0x5F3759DF) - (i >> 1), _F32)
    half = d * 0.5
    for _ in range(3):
        y = y * (1.5 - half * y * y)
    return jnp.where(d > 0.5, y, jnp.zeros_like(y))


def _gcn_body(colp, rowp, x0p,
              s_out, y0, y1, y2, disx,
              acc, dacc,
              cbuf, rbuf, gbuf, onesb, zbuf, z1d,
              d1d, r1d, debuf, xbuf, abuf, sbuf,
              sem0, sem1):
    c = lax.axis_index("c")
    t = lax.axis_index("s")
    coff = c * NPAD
    nbase0 = t * NODES_PER_TILE
    rbase0 = t * ROWS_PER_TILE

    ones16 = jnp.ones((16,), _F32)
    zero16 = jnp.zeros((16,), _F32)

    @pl.loop(0, 8)
    def _c1(v):
        onesb[pl.ds(v * 16, 16)] = ones16

    @pl.loop(0, NCHUNK)
    def _c2(n):
        zbuf[n, :] = zero16

    @pl.loop(0, NCHUNK // 16)
    def _c3(i):
        z1d[pl.ds(i * 16, 16)] = zero16

    @pl.loop(0, NODE_CHUNKS)
    def _zero(q):
        nb = nbase0 + q * NCHUNK
        pltpu.sync_copy(zbuf, acc.at[pl.ds(nb, NCHUNK)])
        pltpu.sync_copy(z1d, dacc.at[pl.ds(nb, NCHUNK)])

    plsc.subcore_barrier()

    @pl.loop(0, EDGE_CHUNKS)
    def _deg(m):
        rb = rbase0 + m * ROWS_PER_CHUNK
        pltpu.sync_copy(colp.at[pl.ds(rb, ROWS_PER_CHUNK)], cbuf)
        copies = [
            pltpu.async_copy(onesb, dacc.at[cbuf.at[r]], sem0, add=True)
            for r in range(ROWS_PER_CHUNK)
        ]
        for d in copies:
            d.wait()

    plsc.subcore_barrier()

    @pl.loop(0, NODE_CHUNKS)
    def _prep(q):
        nb = nbase0 + q * NCHUNK
        pltpu.sync_copy(dacc.at[pl.ds(nb, NCHUNK)], d1d)

        @pl.loop(0, NCHUNK // 16)
        def _r(i):
            d = d1d[pl.ds(i * 16, 16)]
            r1d[pl.ds(i * 16, 16)] = _rsqrt16(d)

        @pl.loop(0, NCHUNK)
        def _b(n):
            idx = jnp.full((16,), n, _I32)
            debuf[n, :] = plsc.load_gather(r1d, [idx])

        pltpu.sync_copy(x0p.at[pl.ds(nb * 32, NCHUNK * 32)], xbuf)

        @pl.loop(0, NCHUNK)
        def _y(n):
            xh = xbuf[pl.ds(n * 32 + c * H, H)]
            sbuf[n, :] = xh
            abuf[n, :] = debuf[n, :] * xh

        pltpu.sync_copy(debuf, disx.at[c, pl.ds(nb, NCHUNK)])
        pltpu.sync_copy(sbuf, s_out.at[pl.ds(coff + nb, NCHUNK)])
        pltpu.sync_copy(abuf, y0.at[pl.ds(coff + nb, NCHUNK)])

    plsc.subcore_barrier()

    for ysrc, ydst in [(y0, y1), (y1, y2), (y2, None)]:
        last = ydst is None

        @pl.loop(0, EDGE_CHUNKS)
        def _edges(m):
            rb = rbase0 + m * ROWS_PER_CHUNK
            pltpu.sync_copy(colp.at[pl.ds(rb, ROWS_PER_CHUNK)], cbuf)

            @pl.loop(0, ROWS_PER_CHUNK)
            def _off(r):
                @pl.loop(0, 8)
                def _offv(v):
                    sl = pl.ds(v * 16, 16)
                    cbuf[r, sl] = cbuf[r, sl] + coff

            pltpu.sync_copy(rowp.at[pl.ds(rb, ROWS_PER_CHUNK)], rbuf)
            gathers = [
                pltpu.async_copy(ysrc.at[cbuf.at[r]], gbuf.at[r], sem0)
                for r in range(ROWS_PER_CHUNK)
            ]
            for d in gathers:
                d.wait()
            scatters = [
                pltpu.async_copy(gbuf.at[r], acc.at[rbuf.at[r]], sem1,
                                 add=True)
                for r in range(ROWS_PER_CHUNK)
            ]
            for d in scatters:
                d.wait()

        plsc.subcore_barrier()

        @pl.loop(0, NODE_CHUNKS)
        def _post(q):
            nb = nbase0 + q * NCHUNK
            pltpu.sync_copy(acc.at[pl.ds(nb, NCHUNK)], abuf)
            if not last:
                pltpu.sync_copy(zbuf, acc.at[pl.ds(nb, NCHUNK)])
            pltpu.sync_copy(disx.at[c, pl.ds(nb, NCHUNK)], debuf)
            pltpu.sync_copy(s_out.at[pl.ds(coff + nb, NCHUNK)], sbuf)

            @pl.loop(0, NCHUNK)
            def _n(n):
                d = debuf[n, :]
                x = d * abuf[n, :]
                if last:
                    sbuf[n, :] = (sbuf[n, :] + x) * 0.25
                else:
                    sbuf[n, :] = sbuf[n, :] + x
                    abuf[n, :] = d * x

            pltpu.sync_copy(sbuf, s_out.at[pl.ds(coff + nb, NCHUNK)])
            if not last:
                pltpu.sync_copy(abuf, ydst.at[pl.ds(coff + nb, NCHUNK)])

        plsc.subcore_barrier()


@jax.jit
def _lightgcn(colp, rowp, x0p):
    mesh = plsc.VectorSubcoreMesh(core_axis_name="c", subcore_axis_name="s",
                                  num_cores=NC, num_subcores=NS)
    f = pl.kernel(
        _gcn_body,
        out_type=(
            jax.ShapeDtypeStruct((2 * NPAD, H), _F32),
            jax.ShapeDtypeStruct((2 * NPAD, H), _F32),
            jax.ShapeDtypeStruct((2 * NPAD, H), _F32),
            jax.ShapeDtypeStruct((2 * NPAD, H), _F32),
            jax.ShapeDtypeStruct((NC, NPAD, H), _F32),
        ),
        mesh=mesh,
        compiler_params=pltpu.CompilerParams(needs_layout_passes=False,
                                             use_tc_tiling_on_sc=False),
        scratch_types=[
            pltpu.VMEM_SHARED((NPAD, H), _F32),
            pltpu.VMEM_SHARED((NPAD,), _F32),
            pltpu.VMEM((ROWS_PER_CHUNK, 128), _I32),
            pltpu.VMEM((ROWS_PER_CHUNK, 128), _I32),
            pltpu.VMEM((ROWS_PER_CHUNK, 128, H), _F32),
            pltpu.VMEM((128,), _F32),
            pltpu.VMEM((NCHUNK, H), _F32),
            pltpu.VMEM((NCHUNK,), _F32),
            pltpu.VMEM((NCHUNK,), _F32),
            pltpu.VMEM((NCHUNK,), _F32),
            pltpu.VMEM((NCHUNK, H), _F32),
            pltpu.VMEM((NCHUNK * 32,), _F32),
            pltpu.VMEM((NCHUNK, H), _F32),
            pltpu.VMEM((NCHUNK, H), _F32),
            pltpu.SemaphoreType.DMA,
            pltpu.SemaphoreType.DMA,
        ],
    )
    return f(colp, rowp, x0p)


def kernel(edge_index, user_weight, item_weight):
    ei = edge_index.astype(_I32)
    pad = N + (jnp.arange(EP - E, dtype=_I32) % 16)
    rowp = jnp.concatenate([ei[0], pad]).reshape(EP // 128, 128)
    colp = jnp.concatenate([ei[1], pad]).reshape(EP // 128, 128)
    x0 = jnp.concatenate([user_weight, item_weight], axis=0)
    x0p = jnp.concatenate(
        [x0, jnp.zeros((NPAD - N, 32), _F32)], axis=0).reshape(NPAD * 32)
    s, _, _, _, _ = _lightgcn(colp, rowp, x0p)
    final = jnp.concatenate([s[:N], s[NPAD:NPAD + N]], axis=1)
    return final[:NUM_USERS], final[NUM_USERS:]
